# baseline (speedup 1.0000x reference)
import functools

import jax
import jax.numpy as jnp
import numpy as np
from jax.experimental import pallas as pl
from jax.experimental.pallas import tpu as pltpu

SEQ = 4096
N1 = 64
F1 = 33
F1P = 40
CH = 128
K = 20
FMAX = SEQ // 2
COLS = N1 * CH


def _tables():
    t = np.arange(N1, dtype=np.float64)
    ang64 = 2.0 * np.pi * np.outer(t, t) / 64.0
    angT = 2.0 * np.pi * np.outer(t, t) / 4096.0
    rep = lambda a: np.repeat(a, CH, axis=1)
    cosF, sinFm = np.cos(ang64), -np.sin(ang64)
    Tc2, Ts2 = rep(np.cos(angT)), rep(-np.sin(angT))
    f1 = np.arange(F1P, dtype=np.float64)
    ang2 = 2.0 * np.pi * np.outer(f1, t) / 64.0
    c2, s2m = np.cos(ang2), -np.sin(ang2)
    c2[F1:, :] = 0.0
    s2m[F1:, :] = 0.0
    ff = 64.0 * f1[:, None] + t[None, :]
    valid = (ff <= FMAX) & (f1[:, None] < F1)
    w = np.where((ff == 0) | (ff == FMAX), 1.0, 2.0) / float(SEQ)
    w = np.where(valid, w, 0.0)
    vt2, wN2 = rep(valid.astype(np.float64)), rep(w)
    ic, is_ = np.cos(ang2).T.copy(), np.sin(ang2).T.copy()
    ic[:, F1:] = 0.0
    is_[:, F1:] = 0.0
    Uc2, Us2 = rep(np.cos(angT)), rep(np.sin(angT))
    Fc, Fs = np.cos(ang64), np.sin(ang64)
    f32 = lambda a: jnp.asarray(a, dtype=jnp.float32)
    names = dict(cosF=cosF, sinFm=sinFm, Tc2=Tc2, Ts2=Ts2, c2=c2, s2m=s2m,
                 vt2=vt2, wN2=wN2, ic=ic, is_=is_, Uc2=Uc2, Us2=Us2,
                 Fc=Fc, Fs=Fs)
    return {k: f32(v) for k, v in names.items()}


TABLE_NAMES = ["cosF", "sinFm", "Tc2", "Ts2", "c2", "s2m", "vt2", "wN2",
               "ic", "is_", "Uc2", "Us2", "Fc", "Fs"]


def _mmh(a, b):
    return jax.lax.dot(a, b, precision=jax.lax.Precision.HIGHEST,
                       preferred_element_type=jnp.float32)


def _mmd(a, b):
    return jax.lax.dot(a, b, precision=jax.lax.Precision.DEFAULT,
                       preferred_element_type=jnp.float32)


def _tp(a):
    return jnp.swapaxes(a.reshape(N1, N1, CH), 0, 1).reshape(N1, COLS)


def _colmax(a):
    r1 = jnp.max(a, axis=0, keepdims=True)
    m = jnp.max(r1.reshape(N1, CH), axis=0, keepdims=True)
    return jnp.broadcast_to(m, (N1, CH)).reshape(1, COLS)


def _fan_kernel(x_ref, cosF, sinFm, Tc2, Ts2, c2, s2m, vt2, wN2,
                ic, is_, Uc2, Us2, Fc, Fs, out_ref):
    x2 = x_ref[0]

    yre = _mmh(cosF[:], x2)
    yim = _mmh(sinFm[:], x2)
    ypre = yre * Tc2[:] - yim * Ts2[:]
    ypim = yre * Ts2[:] + yim * Tc2[:]
    ypre_t = _tp(ypre)
    ypim_t = _tp(ypim)
    xre = _mmh(c2[:], ypre_t) - _mmh(s2m[:], ypim_t)
    xim = _mmh(c2[:], ypim_t) + _mmh(s2m[:], ypre_t)

    mag2 = jnp.where(vt2[:] > 0.0, xre * xre + xim * xim, -1.0)
    work = mag2
    for _ in range(K - 1):
        work = jnp.where(work >= _colmax(work), -2.0, work)
    thr = _colmax(work)
    keep = mag2 >= thr

    coef = jnp.where(keep, wN2[:], 0.0)
    sre = xre * coef
    sim = xim * coef

    zre = _mmd(ic[:], sre) - _mmd(is_[:], sim)
    zim = _mmd(ic[:], sim) + _mmd(is_[:], sre)
    zpre = zre * Uc2[:] - zim * Us2[:]
    zpim = zim * Uc2[:] + zre * Us2[:]
    zpre_t = _tp(zpre)
    zpim_t = _tp(zpim)
    xf = _mmd(Fc[:], zpre_t) - _mmd(Fs[:], zpim_t)

    out_ref[0] = x2 - xf


@jax.jit
def _run(batch_x):
    tabs = _tables()
    B = batch_x.shape[0]
    x2 = batch_x.reshape(B, N1, COLS)
    full = lambda a: pl.BlockSpec(a.shape, lambda b: (0, 0))
    out = pl.pallas_call(
        _fan_kernel,
        grid=(B,),
        in_specs=[pl.BlockSpec((1, N1, COLS), lambda b: (b, 0, 0))]
                 + [full(tabs[n]) for n in TABLE_NAMES],
        out_specs=pl.BlockSpec((1, N1, COLS), lambda b: (b, 0, 0)),
        out_shape=jax.ShapeDtypeStruct((B, N1, COLS), jnp.float32),
        compiler_params=pltpu.CompilerParams(
            dimension_semantics=("parallel",)),
    )(x2, *[tabs[n] for n in TABLE_NAMES])
    return out.reshape(B, SEQ, CH)


def kernel(batch_x, W1, b1, W2, b2, W3, b3):
    return _run(batch_x)


# speedup vs baseline: 1.0894x; 1.0894x over previous
import functools

import jax
import jax.numpy as jnp
import numpy as np
from jax.experimental import pallas as pl
from jax.experimental.pallas import tpu as pltpu

SEQ = 4096
N1 = 64
F1 = 33
F1P = 40
CH = 128
K = 20
FMAX = SEQ // 2
COLS = N1 * CH


def _tables():
    t = np.arange(N1, dtype=np.float64)
    ang64 = 2.0 * np.pi * np.outer(t, t) / 64.0
    angT = 2.0 * np.pi * np.outer(t, t) / 4096.0
    rep = lambda a: np.repeat(a, CH, axis=1)
    cosF, sinFm = np.cos(ang64), -np.sin(ang64)
    Tc2, Ts2 = rep(np.cos(angT)), rep(-np.sin(angT))
    f1 = np.arange(F1P, dtype=np.float64)
    ang2 = 2.0 * np.pi * np.outer(f1, t) / 64.0
    c2, s2m = np.cos(ang2), -np.sin(ang2)
    c2[F1:, :] = 0.0
    s2m[F1:, :] = 0.0
    ff = 64.0 * f1[:, None] + t[None, :]
    valid = (ff <= FMAX) & (f1[:, None] < F1)
    w = np.where((ff == 0) | (ff == FMAX), 1.0, 2.0) / float(SEQ)
    w = np.where(valid, w, 0.0)
    vt2, wN2 = rep(valid.astype(np.float64)), rep(w)
    ic, is_ = np.cos(ang2).T.copy(), np.sin(ang2).T.copy()
    ic[:, F1:] = 0.0
    is_[:, F1:] = 0.0
    Uc2, Us2 = rep(np.cos(angT)), rep(np.sin(angT))
    Fc, Fs = np.cos(ang64), np.sin(ang64)
    f32 = lambda a: jnp.asarray(a, dtype=jnp.float32)
    out = {}
    for k, v in dict(cosF=cosF, sinFm=sinFm, c2=c2, s2m=s2m).items():
        hi = jnp.asarray(v, dtype=jnp.bfloat16)
        lo = (f32(v) - hi.astype(jnp.float32)).astype(jnp.bfloat16)
        out[k + "h"], out[k + "l"] = hi, lo
    for k, v in dict(Tc2=Tc2, Ts2=Ts2, vt2=vt2, wN2=wN2, ic=ic, is_=is_,
                     Uc2=Uc2, Us2=Us2, Fc=Fc, Fs=Fs).items():
        out[k] = f32(v)
    return out


TABLE_NAMES = ["cosFh", "cosFl", "sinFmh", "sinFml", "Tc2", "Ts2",
               "c2h", "c2l", "s2mh", "s2ml", "vt2", "wN2",
               "ic", "is_", "Uc2", "Us2", "Fc", "Fs"]


def _split(a):
    hi = a.astype(jnp.bfloat16)
    lo = (a - hi.astype(jnp.float32)).astype(jnp.bfloat16)
    return hi, lo


def _mm3(wh, wl, bh, bl):
    acc = jax.lax.dot(wh, bh, preferred_element_type=jnp.float32)
    acc += jax.lax.dot(wh, bl, preferred_element_type=jnp.float32)
    acc += jax.lax.dot(wl, bh, preferred_element_type=jnp.float32)
    return acc


def _mmd(a, b):
    return jax.lax.dot(a, b, precision=jax.lax.Precision.DEFAULT,
                       preferred_element_type=jnp.float32)


def _tp(a):
    return jnp.swapaxes(a.reshape(N1, N1, CH), 0, 1).reshape(N1, COLS)


def _colmax(a):
    r1 = jnp.max(a, axis=0, keepdims=True)
    m = jnp.max(r1.reshape(N1, CH), axis=0, keepdims=True)
    return jnp.broadcast_to(m, (N1, CH)).reshape(1, COLS)


def _fan_kernel(x_ref, cosFh, cosFl, sinFmh, sinFml, Tc2, Ts2,
                c2h, c2l, s2mh, s2ml, vt2, wN2,
                ic, is_, Uc2, Us2, Fc, Fs, out_ref):
    x2 = x_ref[0]

    xh, xl = _split(x2)
    yre = _mm3(cosFh[:], cosFl[:], xh, xl)
    yim = _mm3(sinFmh[:], sinFml[:], xh, xl)
    ypre = yre * Tc2[:] - yim * Ts2[:]
    ypim = yre * Ts2[:] + yim * Tc2[:]
    ph, pl = _split(_tp(ypre))
    qh, ql = _split(_tp(ypim))
    xre = _mm3(c2h[:], c2l[:], ph, pl) - _mm3(s2mh[:], s2ml[:], qh, ql)
    xim = _mm3(c2h[:], c2l[:], qh, ql) + _mm3(s2mh[:], s2ml[:], ph, pl)

    mag2 = jnp.where(vt2[:] > 0.0, xre * xre + xim * xim, -1.0)
    work = mag2
    m = _colmax(work)
    for _ in range(K - 1):
        work = jnp.where(work >= m, -2.0, work)
        m = _colmax(work)
    thr = m
    keep = mag2 >= thr

    coef = jnp.where(keep, wN2[:], 0.0)
    sre = xre * coef
    sim = xim * coef

    zre = _mmd(ic[:], sre) - _mmd(is_[:], sim)
    zim = _mmd(ic[:], sim) + _mmd(is_[:], sre)
    zpre = zre * Uc2[:] - zim * Us2[:]
    zpim = zim * Uc2[:] + zre * Us2[:]
    zpre_t = _tp(zpre)
    zpim_t = _tp(zpim)
    xf = _mmd(Fc[:], zpre_t) - _mmd(Fs[:], zpim_t)

    out_ref[0] = x2 - xf


@jax.jit
def _run(batch_x):
    tabs = _tables()
    B = batch_x.shape[0]
    x2 = batch_x.reshape(B, N1, COLS)
    full = lambda a: pl.BlockSpec(a.shape, lambda b: (0, 0))
    out = pl.pallas_call(
        _fan_kernel,
        grid=(B,),
        in_specs=[pl.BlockSpec((1, N1, COLS), lambda b: (b, 0, 0))]
                 + [full(tabs[n]) for n in TABLE_NAMES],
        out_specs=pl.BlockSpec((1, N1, COLS), lambda b: (b, 0, 0)),
        out_shape=jax.ShapeDtypeStruct((B, N1, COLS), jnp.float32),
        compiler_params=pltpu.CompilerParams(
            dimension_semantics=("parallel",)),
    )(x2, *[tabs[n] for n in TABLE_NAMES])
    return out.reshape(B, SEQ, CH)


def kernel(batch_x, W1, b1, W2, b2, W3, b3):
    return _run(batch_x)


# speedup vs baseline: 1.2507x; 1.1481x over previous
import jax
import jax.numpy as jnp
import numpy as np
from jax.experimental import pallas as pl
from jax.experimental.pallas import tpu as pltpu

SEQ = 4096
N1 = 64
F2H = 33
F2P = 40
F1H = 32
CH = 128
K = 20
COLS = N1 * CH


def _tables():
    t = np.arange(N1, dtype=np.float64)
    rep = lambda a: np.repeat(a, CH, axis=1)
    out = {}

    def bsplit(name, a):
        hi = jnp.asarray(a, dtype=jnp.bfloat16)
        lo = (jnp.asarray(a, dtype=jnp.float32)
              - hi.astype(jnp.float32)).astype(jnp.bfloat16)
        out[name + "h"], out[name + "l"] = hi, lo

    f2p = np.arange(F2P, dtype=np.float64)
    ang1 = 2.0 * np.pi * np.outer(f2p, t) / 64.0
    cosF, sinFm = np.cos(ang1), -np.sin(ang1)
    cosF[F2H:, :] = 0.0
    sinFm[F2H:, :] = 0.0
    bsplit("cosF", cosF)
    bsplit("sinFm", sinFm)
    angT = 2.0 * np.pi * np.outer(f2p, t) / 4096.0
    Tc2, Ts2 = np.cos(angT), -np.sin(angT)
    Tc2[F2H:, :] = 0.0
    Ts2[F2H:, :] = 0.0
    out["Tc2"], out["Ts2"] = rep(Tc2), rep(Ts2)
    f1 = np.arange(F1H, dtype=np.float64)
    ang2 = 2.0 * np.pi * np.outer(f1, t) / 64.0
    ang2m = 2.0 * np.pi * np.outer(f1 + 1.0, t) / 64.0
    bsplit("c2", np.cos(ang2))
    bsplit("s2m", -np.sin(ang2))
    bsplit("c2m", np.cos(ang2m))
    bsplit("s2mm", -np.sin(ang2m))
    f2perm = np.where(np.arange(N1) <= 32, np.arange(N1),
                      96 - np.arange(N1)).astype(np.float64)
    ff = 64.0 * f1[:, None] + f2perm[None, :]
    w = np.where(ff == 0, 1.0, 2.0) / float(SEQ)
    out["wN2"] = rep(w)
    out["sgn"] = rep((1.0 - 2.0 * (t % 2))[None, :])
    out["ic"] = np.cos(ang2).T.copy()
    out["is_"] = np.sin(ang2).T.copy()
    angU = 2.0 * np.pi * np.outer(t, f2perm) / 4096.0
    out["Uc2"], out["Us2"] = rep(np.cos(angU)), rep(np.sin(angU))
    angB = 2.0 * np.pi * np.outer(t, f2perm) / 64.0
    out["Fc"], out["Fs"] = np.cos(angB), np.sin(angB)
    f32 = lambda a: (jnp.asarray(a, dtype=jnp.float32)
                     if not isinstance(a, jnp.ndarray) else a)
    return {k: f32(v) for k, v in out.items()}


TABLE_NAMES = ["cosFh", "cosFl", "sinFmh", "sinFml", "Tc2", "Ts2",
               "c2h", "c2l", "s2mh", "s2ml", "c2mh", "c2ml",
               "s2mmh", "s2mml", "wN2", "sgn", "ic", "is_",
               "Uc2", "Us2", "Fc", "Fs"]


def _split(a):
    hi = a.astype(jnp.bfloat16)
    lo = (a - hi.astype(jnp.float32)).astype(jnp.bfloat16)
    return hi, lo


def _mm3(wh, wl, bh, bl):
    acc = jax.lax.dot(wh, bh, preferred_element_type=jnp.float32)
    acc += jax.lax.dot(wh, bl, preferred_element_type=jnp.float32)
    acc += jax.lax.dot(wl, bh, preferred_element_type=jnp.float32)
    return acc


def _mmd(a, b):
    return jax.lax.dot(a, b, precision=jax.lax.Precision.DEFAULT,
                       preferred_element_type=jnp.float32)


def _tp(a, p):
    return jnp.swapaxes(a.reshape(p, N1, CH), 0, 1).reshape(N1, p * CH)


def _colmax(a):
    r1 = jnp.max(a, axis=0, keepdims=True)
    m = jnp.max(r1.reshape(N1, CH), axis=0, keepdims=True)
    return jnp.broadcast_to(m, (N1, CH)).reshape(1, COLS)


def _fan_kernel(x_ref, cosFh, cosFl, sinFmh, sinFml, Tc2, Ts2,
                c2h, c2l, s2mh, s2ml, c2mh, c2ml, s2mmh, s2mml,
                wN2, sgn, ic, is_, Uc2, Us2, Fc, Fs, out_ref):
    x2 = x_ref[0]

    xh, xl = _split(x2)
    yre = _mm3(cosFh[:], cosFl[:], xh, xl)
    yim = _mm3(sinFmh[:], sinFml[:], xh, xl)
    nyq = jnp.sum((yre[0:1] * sgn[:]).reshape(N1, CH),
                  axis=0, keepdims=True)
    ypre = yre * Tc2[:] - yim * Ts2[:]
    ypim = yre * Ts2[:] + yim * Tc2[:]
    ph, pl = _split(_tp(ypre, F2P))
    qh, ql = _split(_tp(ypim, F2P))
    xre_m = _mm3(c2h[:], c2l[:], ph, pl) - _mm3(s2mh[:], s2ml[:], qh, ql)
    xim_m = _mm3(c2h[:], c2l[:], qh, ql) + _mm3(s2mh[:], s2ml[:], ph, pl)
    xre_r = _mm3(c2mh[:], c2ml[:], ph, pl) + _mm3(s2mmh[:], s2mml[:], qh, ql)
    xim_r = _mm3(s2mmh[:], s2mml[:], ph, pl) - _mm3(c2mh[:], c2ml[:], qh, ql)
    xre = jnp.concatenate([xre_m[:, :F2H * CH], xre_r[:, CH:F1H * CH]], axis=1)
    xim = jnp.concatenate([xim_m[:, :F2H * CH], xim_r[:, CH:F1H * CH]], axis=1)

    mag2 = xre * xre + xim * xim
    work = mag2
    m = _colmax(work)
    for _ in range(K - 1):
        work = jnp.where(work >= m, -2.0, work)
        s_prev, m = m, _colmax(work)
    s19, s20 = s_prev, m
    nyq2 = nyq * nyq
    nyq2x = jnp.broadcast_to(nyq2, (N1, CH)).reshape(1, COLS)
    keepn = nyq2x > s20
    thr = jnp.where(keepn, s19, s20)
    keep = mag2 >= thr

    coef = jnp.where(keep, wN2[:], 0.0)
    sre = xre * coef
    sim = xim * coef

    zre = _mmd(ic[:], sre) - _mmd(is_[:], sim)
    zim = _mmd(ic[:], sim) + _mmd(is_[:], sre)
    zpre = zre * Uc2[:] - zim * Us2[:]
    zpim = zim * Uc2[:] + zre * Us2[:]
    zpre_t = _tp(zpre, N1)
    zpim_t = _tp(zpim, N1)
    xf = _mmd(Fc[:], zpre_t) - _mmd(Fs[:], zpim_t)
    nv = jnp.where(keepn[:, :CH], nyq * (1.0 / SEQ), 0.0)
    nvx = jnp.broadcast_to(nv, (N1, CH)).reshape(1, COLS)
    xf = xf + sgn[:] * nvx

    out_ref[0] = x2 - xf


@jax.jit
def _run(batch_x):
    tabs = _tables()
    B = batch_x.shape[0]
    x2 = batch_x.reshape(B, N1, COLS)
    full = lambda a: pl.BlockSpec(a.shape, lambda b: (0, 0))
    out = pl.pallas_call(
        _fan_kernel,
        grid=(B,),
        in_specs=[pl.BlockSpec((1, N1, COLS), lambda b: (b, 0, 0))]
                 + [full(tabs[n]) for n in TABLE_NAMES],
        out_specs=pl.BlockSpec((1, N1, COLS), lambda b: (b, 0, 0)),
        out_shape=jax.ShapeDtypeStruct((B, N1, COLS), jnp.float32),
        compiler_params=pltpu.CompilerParams(
            dimension_semantics=("arbitrary",)),
    )(x2, *[tabs[n] for n in TABLE_NAMES])
    return out.reshape(B, SEQ, CH)


def kernel(batch_x, W1, b1, W2, b2, W3, b3):
    return _run(batch_x)


# speedup vs baseline: 1.3515x; 1.0805x over previous
import jax
import jax.numpy as jnp
import numpy as np
from jax.experimental import pallas as pl
from jax.experimental.pallas import tpu as pltpu

SEQ = 4096
N1 = 64
F2H = 33
F2P = 40
F1H = 32
CH = 128
K = 20
COLS = N1 * CH


def _tables():
    t = np.arange(N1, dtype=np.float64)
    rep = lambda a: np.repeat(a, CH, axis=1)
    out = {}

    def bsplit(name, a):
        hi = jnp.asarray(a, dtype=jnp.bfloat16)
        lo = (jnp.asarray(a, dtype=jnp.float32)
              - hi.astype(jnp.float32)).astype(jnp.bfloat16)
        out[name + "h"], out[name + "l"] = hi, lo

    f2p = np.arange(F2P, dtype=np.float64)
    ang1 = 2.0 * np.pi * np.outer(f2p, t) / 64.0
    cosF, sinFm = np.cos(ang1), -np.sin(ang1)
    cosF[F2H:, :] = 0.0
    sinFm[F2H:, :] = 0.0
    bsplit("L1", np.concatenate([cosF, sinFm], axis=0))
    angT = 2.0 * np.pi * np.outer(f2p, t) / 4096.0
    Tc2, Ts2 = np.cos(angT), -np.sin(angT)
    Tc2[F2H:, :] = 0.0
    Ts2[F2H:, :] = 0.0
    out["Tc2"], out["Ts2"] = rep(Tc2), rep(Ts2)
    f1 = np.arange(F1H, dtype=np.float64)
    ang2 = 2.0 * np.pi * np.outer(f1, t) / 64.0
    ang2m = 2.0 * np.pi * np.outer(f1 + 1.0, t) / 64.0
    bsplit("L2", np.concatenate([np.cos(ang2), -np.sin(ang2),
                                 np.cos(ang2m), -np.sin(ang2m)],
                                axis=0))
    f2perm = np.where(np.arange(N1) <= 32, np.arange(N1),
                      96 - np.arange(N1)).astype(np.float64)
    ff = 64.0 * f1[:, None] + f2perm[None, :]
    w = np.where(ff == 0, 1.0, 2.0) / float(SEQ)
    out["wN2"] = rep(w)
    out["sgn"] = rep((1.0 - 2.0 * (t % 2))[None, :])
    out["LA"] = np.concatenate([np.cos(ang2).T, np.sin(ang2).T], axis=0)
    angU = 2.0 * np.pi * np.outer(t, f2perm) / 4096.0
    out["Uc2"], out["Us2"] = rep(np.cos(angU)), rep(np.sin(angU))
    angB = 2.0 * np.pi * np.outer(t, f2perm) / 64.0
    out["Fcb"] = jnp.asarray(np.cos(angB), dtype=jnp.bfloat16)
    out["Fsb"] = jnp.asarray(np.sin(angB), dtype=jnp.bfloat16)
    f32 = lambda a: (jnp.asarray(a, dtype=jnp.float32)
                     if not isinstance(a, jnp.ndarray) else a)
    return {k: f32(v) for k, v in out.items()}


TABLE_NAMES = ["L1h", "L1l", "Tc2", "Ts2", "L2h", "L2l",
               "wN2", "sgn", "LA", "Uc2", "Us2", "Fcb", "Fsb"]


def _split(a):
    hi = a.astype(jnp.bfloat16)
    lo = (a - hi.astype(jnp.float32)).astype(jnp.bfloat16)
    return hi, lo


def _mm3(wh, wl, bh, bl):
    acc = jax.lax.dot(wh, bh, preferred_element_type=jnp.float32)
    acc += jax.lax.dot(wh, bl, preferred_element_type=jnp.float32)
    acc += jax.lax.dot(wl, bh, preferred_element_type=jnp.float32)
    return acc


def _mmd(a, b):
    return jax.lax.dot(a, b, precision=jax.lax.Precision.DEFAULT,
                       preferred_element_type=jnp.float32)


def _tp(a, p):
    return jnp.swapaxes(a.reshape(p, N1, CH), 0, 1).reshape(N1, p * CH)


def _colmax(a):
    r1 = jnp.max(a, axis=0, keepdims=True)
    m = jnp.max(r1.reshape(N1, CH), axis=0, keepdims=True)
    return jnp.broadcast_to(m, (N1, CH)).reshape(1, COLS)


def _fan_kernel(x_ref, L1h, L1l, Tc2, Ts2, L2h, L2l,
                wN2, sgn, LA, Uc2, Us2, Fcb, Fsb, out_ref):
    x2 = x_ref[0]

    xh, xl = _split(x2)
    y = _mm3(L1h[:], L1l[:], xh, xl)
    yre, yim = y[:F2P], y[F2P:]
    nyq = jnp.sum((yre[0:1] * sgn[:]).reshape(N1, CH),
                  axis=0, keepdims=True)
    ypre = yre * Tc2[:] - yim * Ts2[:]
    ypim = yre * Ts2[:] + yim * Tc2[:]
    ph, pl = _split(_tp(ypre, F2P))
    qh, ql = _split(_tp(ypim, F2P))
    P = _mm3(L2h[:], L2l[:], ph, pl)
    Q = _mm3(L2h[:], L2l[:], qh, ql)
    xre_m = P[0:32] - Q[32:64]
    xim_m = Q[0:32] + P[32:64]
    xre_r = P[64:96] + Q[96:128]
    xim_r = P[96:128] - Q[64:96]
    xre = jnp.concatenate([xre_m[:, :F2H * CH], xre_r[:, CH:F1H * CH]], axis=1)
    xim = jnp.concatenate([xim_m[:, :F2H * CH], xim_r[:, CH:F1H * CH]], axis=1)

    mag2 = xre * xre + xim * xim
    work = mag2
    m = _colmax(work)
    for _ in range(K - 1):
        work = jnp.where(work >= m, -2.0, work)
        s_prev, m = m, _colmax(work)
    s19, s20 = s_prev, m
    nyq2 = nyq * nyq
    nyq2x = jnp.broadcast_to(nyq2, (N1, CH)).reshape(1, COLS)
    keepn = nyq2x > s20
    thr = jnp.where(keepn, s19, s20)
    keep = mag2 >= thr

    coef = jnp.where(keep, wN2[:], 0.0)
    sre = xre * coef
    sim = xim * coef

    U = _mmd(LA[:], sre)
    V = _mmd(LA[:], sim)
    zre = U[:N1] - V[N1:]
    zim = V[:N1] + U[N1:]
    zpre = zre * Uc2[:] - zim * Us2[:]
    zpim = zim * Uc2[:] + zre * Us2[:]
    zpre_t = _tp(zpre.astype(jnp.bfloat16), N1)
    zpim_t = _tp(zpim.astype(jnp.bfloat16), N1)
    xf = (jax.lax.dot(Fcb[:], zpre_t, preferred_element_type=jnp.float32)
          - jax.lax.dot(Fsb[:], zpim_t,
                        preferred_element_type=jnp.float32))
    nv = jnp.where(keepn[:, :CH], nyq * (1.0 / SEQ), 0.0)
    nvx = jnp.broadcast_to(nv, (N1, CH)).reshape(1, COLS)
    xf = xf + sgn[:] * nvx

    out_ref[0] = x2 - xf


@jax.jit
def _run(batch_x):
    tabs = _tables()
    B = batch_x.shape[0]
    x2 = batch_x.reshape(B, N1, COLS)
    full = lambda a: pl.BlockSpec(a.shape, lambda b: (0, 0))
    out = pl.pallas_call(
        _fan_kernel,
        grid=(B,),
        in_specs=[pl.BlockSpec((1, N1, COLS), lambda b: (b, 0, 0))]
                 + [full(tabs[n]) for n in TABLE_NAMES],
        out_specs=pl.BlockSpec((1, N1, COLS), lambda b: (b, 0, 0)),
        out_shape=jax.ShapeDtypeStruct((B, N1, COLS), jnp.float32),
        compiler_params=pltpu.CompilerParams(
            dimension_semantics=("arbitrary",)),
    )(x2, *[tabs[n] for n in TABLE_NAMES])
    return out.reshape(B, SEQ, CH)


def kernel(batch_x, W1, b1, W2, b2, W3, b3):
    return _run(batch_x)


# speedup vs baseline: 1.3557x; 1.0032x over previous
import jax
import jax.numpy as jnp
import numpy as np
from jax.experimental import pallas as pl
from jax.experimental.pallas import tpu as pltpu

SEQ = 4096
N1 = 64
F2H = 33
F2P = 40
F1H = 32
CH = 128
K = 20
COLS = N1 * CH


def _tables():
    t = np.arange(N1, dtype=np.float64)
    rep = lambda a: np.repeat(a, CH, axis=1)
    out = {}

    def bsplit(name, a):
        hi = jnp.asarray(a, dtype=jnp.bfloat16)
        lo = (jnp.asarray(a, dtype=jnp.float32)
              - hi.astype(jnp.float32)).astype(jnp.bfloat16)
        out[name + "h"], out[name + "l"] = hi, lo

    f2p = np.arange(F2P, dtype=np.float64)
    ang1 = 2.0 * np.pi * np.outer(f2p, t) / 64.0
    cosF, sinFm = np.cos(ang1), -np.sin(ang1)
    cosF[F2H:, :] = 0.0
    sinFm[F2H:, :] = 0.0
    bsplit("L1", np.concatenate([cosF, sinFm], axis=0))
    angT = 2.0 * np.pi * np.outer(f2p, t) / 4096.0
    Tc2, Ts2 = np.cos(angT), -np.sin(angT)
    Tc2[F2H:, :] = 0.0
    Ts2[F2H:, :] = 0.0
    out["Tc2"], out["Ts2"] = rep(Tc2), rep(Ts2)
    f1 = np.arange(F1H, dtype=np.float64)
    ang2 = 2.0 * np.pi * np.outer(f1, t) / 64.0
    ang2m = 2.0 * np.pi * np.outer(f1 + 1.0, t) / 64.0
    bsplit("L2", np.concatenate([np.cos(ang2), -np.sin(ang2),
                                 np.cos(ang2m), -np.sin(ang2m)],
                                axis=0))
    f2perm = np.where(np.arange(N1) <= 32, np.arange(N1),
                      96 - np.arange(N1)).astype(np.float64)
    ff = 64.0 * f1[:, None] + f2perm[None, :]
    w = np.where(ff == 0, 1.0, 2.0) / float(SEQ)
    out["wN2"] = rep(w)
    out["sgn"] = rep((1.0 - 2.0 * (t % 2))[None, :])
    out["LA"] = np.concatenate([np.cos(ang2).T, np.sin(ang2).T], axis=0)
    angU = 2.0 * np.pi * np.outer(t, f2perm) / 4096.0
    out["Uc2"], out["Us2"] = rep(np.cos(angU)), rep(np.sin(angU))
    angB = 2.0 * np.pi * np.outer(t, f2perm) / 64.0
    out["Fcb"] = jnp.asarray(np.cos(angB), dtype=jnp.bfloat16)
    out["Fsb"] = jnp.asarray(np.sin(angB), dtype=jnp.bfloat16)
    f32 = lambda a: (jnp.asarray(a, dtype=jnp.float32)
                     if not isinstance(a, jnp.ndarray) else a)
    return {k: f32(v) for k, v in out.items()}


TABLE_NAMES = ["L1h", "L1l", "Tc2", "Ts2", "L2h", "L2l",
               "wN2", "sgn", "LA", "Uc2", "Us2", "Fcb", "Fsb"]


def _split(a):
    hi = a.astype(jnp.bfloat16)
    lo = (a - hi.astype(jnp.float32)).astype(jnp.bfloat16)
    return hi, lo


def _mm3(wh, wl, bh, bl):
    acc = jax.lax.dot(wh, bh, preferred_element_type=jnp.float32)
    acc += jax.lax.dot(wh, bl, preferred_element_type=jnp.float32)
    acc += jax.lax.dot(wl, bh, preferred_element_type=jnp.float32)
    return acc


def _mmd(a, b):
    return jax.lax.dot(a, b, precision=jax.lax.Precision.DEFAULT,
                       preferred_element_type=jnp.float32)


def _tp(a, p):
    return jnp.swapaxes(a.reshape(p, N1, CH), 0, 1).reshape(N1, p * CH)


def _colmax(a):
    r1 = jnp.max(a, axis=0, keepdims=True)
    m = jnp.max(r1.reshape(N1, CH), axis=0, keepdims=True)
    return jnp.broadcast_to(m, (N1, CH)).reshape(1, COLS)


def _fan_kernel(x_ref, L1h, L1l, Tc2, Ts2, L2h, L2l,
                wN2, sgn, LA, Uc2, Us2, Fcb, Fsb, out_ref):
    x2 = x_ref[0]

    xh, xl = _split(x2)
    y = _mm3(L1h[:], L1l[:], xh, xl)
    yre, yim = y[:F2P], y[F2P:]
    nyq = jnp.sum((yre[0:1] * sgn[:]).reshape(N1, CH),
                  axis=0, keepdims=True)
    ypre = yre * Tc2[:] - yim * Ts2[:]
    ypim = yre * Ts2[:] + yim * Tc2[:]
    ph, pl = _split(_tp(ypre, F2P))
    qh, ql = _split(_tp(ypim, F2P))
    P = _mm3(L2h[:], L2l[:], ph, pl)
    Q = _mm3(L2h[:], L2l[:], qh, ql)
    xre_m = P[0:32] - Q[32:64]
    xim_m = Q[0:32] + P[32:64]
    xre_r = P[64:96] + Q[96:128]
    xim_r = P[96:128] - Q[64:96]
    xre = jnp.concatenate([xre_m[:, :F2H * CH], xre_r[:, CH:F1H * CH]], axis=1)
    xim = jnp.concatenate([xim_m[:, :F2H * CH], xim_r[:, CH:F1H * CH]], axis=1)

    mag2 = xre * xre + xim * xim
    top = jnp.maximum(mag2[:16], mag2[16:])
    bot = jnp.minimum(mag2[:16], mag2[16:])
    m = _colmax(top)
    for _ in range(K - 1):
        hit = top >= m
        top, bot = (jnp.where(hit, bot, top),
                    jnp.where(hit, -2.0, bot))
        s_prev, m = m, _colmax(top)
    s19, s20 = s_prev, m
    nyq2 = nyq * nyq
    nyq2x = jnp.broadcast_to(nyq2, (N1, CH)).reshape(1, COLS)
    keepn = nyq2x > s20
    thr = jnp.where(keepn, s19, s20)
    keep = mag2 >= thr

    coef = jnp.where(keep, wN2[:], 0.0)
    sre = xre * coef
    sim = xim * coef

    U = _mmd(LA[:], sre)
    V = _mmd(LA[:], sim)
    zre = U[:N1] - V[N1:]
    zim = V[:N1] + U[N1:]
    zpre = zre * Uc2[:] - zim * Us2[:]
    zpim = zim * Uc2[:] + zre * Us2[:]
    zpre_t = _tp(zpre.astype(jnp.bfloat16), N1)
    zpim_t = _tp(zpim.astype(jnp.bfloat16), N1)
    xf = (jax.lax.dot(Fcb[:], zpre_t, preferred_element_type=jnp.float32)
          - jax.lax.dot(Fsb[:], zpim_t,
                        preferred_element_type=jnp.float32))
    nv = jnp.where(keepn[:, :CH], nyq * (1.0 / SEQ), 0.0)
    nvx = jnp.broadcast_to(nv, (N1, CH)).reshape(1, COLS)
    xf = xf + sgn[:] * nvx

    out_ref[0] = x2 - xf


@jax.jit
def _run(batch_x):
    tabs = _tables()
    B = batch_x.shape[0]
    x2 = batch_x.reshape(B, N1, COLS)
    full = lambda a: pl.BlockSpec(a.shape, lambda b: (0, 0))
    out = pl.pallas_call(
        _fan_kernel,
        grid=(B,),
        in_specs=[pl.BlockSpec((1, N1, COLS), lambda b: (b, 0, 0))]
                 + [full(tabs[n]) for n in TABLE_NAMES],
        out_specs=pl.BlockSpec((1, N1, COLS), lambda b: (b, 0, 0)),
        out_shape=jax.ShapeDtypeStruct((B, N1, COLS), jnp.float32),
        compiler_params=pltpu.CompilerParams(
            dimension_semantics=("arbitrary",)),
    )(x2, *[tabs[n] for n in TABLE_NAMES])
    return out.reshape(B, SEQ, CH)


def kernel(batch_x, W1, b1, W2, b2, W3, b3):
    return _run(batch_x)
